# Initial kernel scaffold; baseline (speedup 1.0000x reference)
import functools

import jax
import jax.numpy as jnp
from jax import lax
from jax.experimental import pallas as pl
from jax.experimental.pallas import tpu as pltpu
from jax.experimental.pallas import tpu_sc as plsc

N_NODES = 10000
N_EDGES = 320000
IN_CH = 128
HIDDEN = 128
OUT_CH = 64

NC = 2
NS = 16
NT = NC * NS
CHUNK = 128
EPT = 10240
E_PAD = NT * EPT
BLKS_PER_TILE = EPT // CHUNK
ACC_ROWS = 10240
ROWS_PER_TILE = ACC_ROWS // NS


def _make_edge_agg(feat):
    mesh = plsc.VectorSubcoreMesh(core_axis_name="c", subcore_axis_name="s")

    @functools.partial(
        pl.kernel,
        out_type=jax.ShapeDtypeStruct((NC, ACC_ROWS, feat), jnp.float32),
        mesh=mesh,
        scratch_types=[
            pltpu.VMEM((BLKS_PER_TILE, CHUNK), jnp.int32),
            pltpu.VMEM((BLKS_PER_TILE, CHUNK), jnp.int32),
            pltpu.VMEM((CHUNK, feat), jnp.float32),
            pltpu.VMEM_SHARED((ACC_ROWS, feat), jnp.float32),
            pltpu.SemaphoreType.DMA,
        ],
    )
    def edge_agg(z_hbm, src_hbm, dst_hbm, zero_hbm, out_hbm,
                 src_v, dst_v, rows_v, acc_sh, sem):
        cid = lax.axis_index("c")
        sid = lax.axis_index("s")
        wid = cid * NS + sid

        r0 = sid * ROWS_PER_TILE
        pltpu.sync_copy(zero_hbm.at[pl.ds(r0, ROWS_PER_TILE)],
                        acc_sh.at[pl.ds(r0, ROWS_PER_TILE)])
        b0 = wid * BLKS_PER_TILE
        pltpu.sync_copy(src_hbm.at[pl.ds(b0, BLKS_PER_TILE)], src_v)
        pltpu.sync_copy(dst_hbm.at[pl.ds(b0, BLKS_PER_TILE)], dst_v)
        plsc.subcore_barrier()

        def body(j, carry):
            pltpu.async_copy(z_hbm.at[src_v.at[j]], rows_v, sem).wait()
            pltpu.sync_copy(rows_v, acc_sh.at[dst_v.at[j]], add=True)
            return carry

        lax.fori_loop(0, BLKS_PER_TILE, body, 0)
        plsc.subcore_barrier()
        pltpu.sync_copy(acc_sh.at[pl.ds(r0, ROWS_PER_TILE)],
                        out_hbm.at[cid, pl.ds(r0, ROWS_PER_TILE)])

    return edge_agg


_edge_agg_128 = _make_edge_agg(HIDDEN)
_edge_agg_64 = _make_edge_agg(OUT_CH)

ROW_BLK = 1000
GRID = N_NODES // ROW_BLK


def _tc_a_body(x_ref, wl_ref, wr_ref, b_ref, z_ref, r_ref):
    xv = x_ref[...]
    z_ref[...] = jnp.dot(xv, wl_ref[...], preferred_element_type=jnp.float32)
    r_ref[...] = (jnp.dot(xv, wr_ref[...], preferred_element_type=jnp.float32)
                  + b_ref[...])


def _tc_b_body(p0_ref, p1_ref, r1_ref, wl_ref, wr_ref, b_ref, z_ref, r_ref):
    h = jnp.maximum(p0_ref[0] + p1_ref[0] + r1_ref[...], 0.0)
    z_ref[...] = jnp.dot(h, wl_ref[...], preferred_element_type=jnp.float32)
    r_ref[...] = (jnp.dot(h, wr_ref[...], preferred_element_type=jnp.float32)
                  + b_ref[...])


def _tc_c_body(q0_ref, q1_ref, r2_ref, o_ref):
    v = q0_ref[0] + q1_ref[0] + r2_ref[...]
    m = jnp.max(v, axis=1, keepdims=True)
    e = jnp.exp(v - m)
    o_ref[...] = e / jnp.sum(e, axis=1, keepdims=True)


def _tc_a(x, wlT, wrT, b):
    return pl.pallas_call(
        _tc_a_body,
        grid=(GRID,),
        in_specs=[
            pl.BlockSpec((ROW_BLK, IN_CH), lambda i: (i, 0)),
            pl.BlockSpec((IN_CH, HIDDEN), lambda i: (0, 0)),
            pl.BlockSpec((IN_CH, HIDDEN), lambda i: (0, 0)),
            pl.BlockSpec((1, HIDDEN), lambda i: (0, 0)),
        ],
        out_specs=[
            pl.BlockSpec((ROW_BLK, HIDDEN), lambda i: (i, 0)),
            pl.BlockSpec((ROW_BLK, HIDDEN), lambda i: (i, 0)),
        ],
        out_shape=[
            jax.ShapeDtypeStruct((N_NODES, HIDDEN), jnp.float32),
            jax.ShapeDtypeStruct((N_NODES, HIDDEN), jnp.float32),
        ],
    )(x, wlT, wrT, b)


def _tc_b(p, r1, wlT, wrT, b):
    return pl.pallas_call(
        _tc_b_body,
        grid=(GRID,),
        in_specs=[
            pl.BlockSpec((1, ROW_BLK, HIDDEN), lambda i: (0, i, 0)),
            pl.BlockSpec((1, ROW_BLK, HIDDEN), lambda i: (1, i, 0)),
            pl.BlockSpec((ROW_BLK, HIDDEN), lambda i: (i, 0)),
            pl.BlockSpec((HIDDEN, OUT_CH), lambda i: (0, 0)),
            pl.BlockSpec((HIDDEN, OUT_CH), lambda i: (0, 0)),
            pl.BlockSpec((1, OUT_CH), lambda i: (0, 0)),
        ],
        out_specs=[
            pl.BlockSpec((ROW_BLK, OUT_CH), lambda i: (i, 0)),
            pl.BlockSpec((ROW_BLK, OUT_CH), lambda i: (i, 0)),
        ],
        out_shape=[
            jax.ShapeDtypeStruct((N_NODES, OUT_CH), jnp.float32),
            jax.ShapeDtypeStruct((N_NODES, OUT_CH), jnp.float32),
        ],
    )(p, p, r1, wlT, wrT, b)


def _tc_c(q, r2):
    return pl.pallas_call(
        _tc_c_body,
        grid=(GRID,),
        in_specs=[
            pl.BlockSpec((1, ROW_BLK, OUT_CH), lambda i: (0, i, 0)),
            pl.BlockSpec((1, ROW_BLK, OUT_CH), lambda i: (1, i, 0)),
            pl.BlockSpec((ROW_BLK, OUT_CH), lambda i: (i, 0)),
        ],
        out_specs=pl.BlockSpec((ROW_BLK, OUT_CH), lambda i: (i, 0)),
        out_shape=jax.ShapeDtypeStruct((N_NODES, OUT_CH), jnp.float32),
    )(q, q, r2)


def kernel(x, edge_index, W1_l, b1_l, W1_r, W2_l, b2_l, W2_r):
    src = edge_index[0].astype(jnp.int32)
    dst = edge_index[1].astype(jnp.int32)
    pad = E_PAD - N_EDGES
    src_p = jnp.concatenate([src, jnp.zeros((pad,), jnp.int32)])
    dst_p = jnp.concatenate([dst, jnp.full((pad,), N_NODES, jnp.int32)])
    src_p = src_p.reshape(E_PAD // CHUNK, CHUNK)
    dst_p = dst_p.reshape(E_PAD // CHUNK, CHUNK)

    z1, r1 = _tc_a(x, W1_l.T, W1_r.T, b1_l.reshape(1, HIDDEN))
    zero128 = jnp.zeros((ACC_ROWS, HIDDEN), jnp.float32)
    p = _edge_agg_128(z1, src_p, dst_p, zero128)
    z2, r2 = _tc_b(p, r1, W2_l.T, W2_r.T, b2_l.reshape(1, OUT_CH))
    zero64 = jnp.zeros((ACC_ROWS, OUT_CH), jnp.float32)
    q = _edge_agg_64(z2, src_p, dst_p, zero64)
    return _tc_c(q, r2)


# speedup vs baseline: 2.8137x; 2.8137x over previous
import functools

import jax
import jax.numpy as jnp
from jax import lax
from jax.experimental import pallas as pl
from jax.experimental.pallas import tpu as pltpu
from jax.experimental.pallas import tpu_sc as plsc

N_NODES = 10000
N_EDGES = 320000
IN_CH = 128
HIDDEN = 128
OUT_CH = 64

NC = 2
NS = 16
NT = NC * NS
CHUNK = 128
EPT = 10240
E_PAD = NT * EPT
BLKS_PER_TILE = EPT // CHUNK
ACC_ROWS = 10240
ROWS_PER_TILE = ACC_ROWS // NS


@functools.lru_cache(maxsize=None)
def _make_edge_agg(feat):
    mesh = plsc.VectorSubcoreMesh(core_axis_name="c", subcore_axis_name="s",
                                  num_cores=NC, num_subcores=NS)

    @functools.partial(
        pl.kernel,
        out_type=jax.ShapeDtypeStruct((NC, ACC_ROWS, feat), jnp.float32),
        mesh=mesh,
        scratch_types=[
            pltpu.VMEM((BLKS_PER_TILE, CHUNK), jnp.int32),
            pltpu.VMEM((BLKS_PER_TILE, CHUNK), jnp.int32),
            pltpu.VMEM((CHUNK, feat), jnp.float32),
            pltpu.VMEM_SHARED((ACC_ROWS, feat), jnp.float32),
            pltpu.SemaphoreType.DMA,
        ],
        compiler_params=pltpu.CompilerParams(use_tc_tiling_on_sc=False),
    )
    def edge_agg(z_hbm, src_hbm, dst_hbm, zero_hbm, out_hbm,
                 src_v, dst_v, rows_v, acc_sh, sem):
        cid = lax.axis_index("c")
        sid = lax.axis_index("s")
        wid = cid * NS + sid

        r0 = sid * ROWS_PER_TILE
        pltpu.sync_copy(zero_hbm.at[pl.ds(r0, ROWS_PER_TILE)],
                        acc_sh.at[pl.ds(r0, ROWS_PER_TILE)])
        b0 = wid * BLKS_PER_TILE
        pltpu.sync_copy(src_hbm.at[pl.ds(b0, BLKS_PER_TILE)], src_v)
        pltpu.sync_copy(dst_hbm.at[pl.ds(b0, BLKS_PER_TILE)], dst_v)
        plsc.subcore_barrier()

        def body(j, carry):
            pltpu.async_copy(z_hbm.at[src_v.at[j]], rows_v, sem).wait()
            pltpu.sync_copy(rows_v, acc_sh.at[dst_v.at[j]], add=True)
            return carry

        lax.fori_loop(0, BLKS_PER_TILE, body, 0)
        plsc.subcore_barrier()
        pltpu.sync_copy(acc_sh.at[pl.ds(r0, ROWS_PER_TILE)],
                        out_hbm.at[cid, pl.ds(r0, ROWS_PER_TILE)])

    return edge_agg


ROW_BLK = 1000
GRID = N_NODES // ROW_BLK


def _layer1_body(p0_ref, p1_ref, x_ref, wl_ref, wr_ref, b_ref, h_ref):
    agg = p0_ref[0] + p1_ref[0]
    pre = (jnp.dot(agg, wl_ref[...], preferred_element_type=jnp.float32)
           + b_ref[...]
           + jnp.dot(x_ref[...], wr_ref[...],
                     preferred_element_type=jnp.float32))
    h_ref[...] = jnp.maximum(pre, 0.0)


def _layer2_body(q0_ref, q1_ref, h_ref, wl_ref, wr_ref, b_ref, o_ref):
    agg = q0_ref[0] + q1_ref[0]
    v = (jnp.dot(agg, wl_ref[...], preferred_element_type=jnp.float32)
         + b_ref[...]
         + jnp.dot(h_ref[...], wr_ref[...],
                   preferred_element_type=jnp.float32))
    m = jnp.max(v, axis=1, keepdims=True)
    e = jnp.exp(v - m)
    o_ref[...] = e / jnp.sum(e, axis=1, keepdims=True)


def _layer1(p, x, wlT, wrT, b):
    return pl.pallas_call(
        _layer1_body,
        grid=(GRID,),
        in_specs=[
            pl.BlockSpec((1, ROW_BLK, IN_CH), lambda i: (0, i, 0)),
            pl.BlockSpec((1, ROW_BLK, IN_CH), lambda i: (1, i, 0)),
            pl.BlockSpec((ROW_BLK, IN_CH), lambda i: (i, 0)),
            pl.BlockSpec((IN_CH, HIDDEN), lambda i: (0, 0)),
            pl.BlockSpec((IN_CH, HIDDEN), lambda i: (0, 0)),
            pl.BlockSpec((1, HIDDEN), lambda i: (0, 0)),
        ],
        out_specs=pl.BlockSpec((ROW_BLK, HIDDEN), lambda i: (i, 0)),
        out_shape=jax.ShapeDtypeStruct((N_NODES, HIDDEN), jnp.float32),
    )(p, p, x, wlT, wrT, b)


def _layer2(q, h, wlT, wrT, b):
    return pl.pallas_call(
        _layer2_body,
        grid=(GRID,),
        in_specs=[
            pl.BlockSpec((1, ROW_BLK, HIDDEN), lambda i: (0, i, 0)),
            pl.BlockSpec((1, ROW_BLK, HIDDEN), lambda i: (1, i, 0)),
            pl.BlockSpec((ROW_BLK, HIDDEN), lambda i: (i, 0)),
            pl.BlockSpec((HIDDEN, OUT_CH), lambda i: (0, 0)),
            pl.BlockSpec((HIDDEN, OUT_CH), lambda i: (0, 0)),
            pl.BlockSpec((1, OUT_CH), lambda i: (0, 0)),
        ],
        out_specs=pl.BlockSpec((ROW_BLK, OUT_CH), lambda i: (i, 0)),
        out_shape=jax.ShapeDtypeStruct((N_NODES, OUT_CH), jnp.float32),
    )(q, q, h, wlT, wrT, b)


def kernel(x, edge_index, W1_l, b1_l, W1_r, W2_l, b2_l, W2_r):
    src = edge_index[0].astype(jnp.int32)
    dst = edge_index[1].astype(jnp.int32)
    pad = E_PAD - N_EDGES
    src_p = jnp.concatenate([src, jnp.zeros((pad,), jnp.int32)])
    dst_p = jnp.concatenate([dst, jnp.full((pad,), N_NODES, jnp.int32)])
    src_p = src_p.reshape(E_PAD // CHUNK, CHUNK)
    dst_p = dst_p.reshape(E_PAD // CHUNK, CHUNK)

    zero128 = jnp.zeros((ACC_ROWS, IN_CH), jnp.float32)
    p = _make_edge_agg(IN_CH)(x, src_p, dst_p, zero128)
    h = _layer1(p, x, W1_l.T, W1_r.T, b1_l.reshape(1, HIDDEN))
    q = _make_edge_agg(HIDDEN)(h, src_p, dst_p, zero128)
    return _layer2(q, h, W2_l.T, W2_r.T, b2_l.reshape(1, OUT_CH))


# speedup vs baseline: 2.8768x; 1.0224x over previous
import functools

import jax
import jax.numpy as jnp
from jax import lax
from jax.experimental import pallas as pl
from jax.experimental.pallas import tpu as pltpu
from jax.experimental.pallas import tpu_sc as plsc

N_NODES = 10000
N_EDGES = 320000
IN_CH = 128
HIDDEN = 128
OUT_CH = 64

NC = 2
NS = 16
NT = NC * NS
EPT = 10240
E_PAD = NT * EPT
ACC_ROWS = 10240
ROWS_PER_TILE = ACC_ROWS // NS


@functools.lru_cache(maxsize=None)
def _make_edge_agg(feat, chunk):
    blks = EPT // chunk
    mesh = plsc.VectorSubcoreMesh(core_axis_name="c", subcore_axis_name="s",
                                  num_cores=NC, num_subcores=NS)

    @functools.partial(
        pl.kernel,
        out_type=jax.ShapeDtypeStruct((NC, ACC_ROWS, feat), jnp.float32),
        mesh=mesh,
        scratch_types=[
            pltpu.VMEM((blks, chunk), jnp.int32),
            pltpu.VMEM((blks, chunk), jnp.int32),
            pltpu.VMEM((2, chunk, feat), jnp.float32),
            pltpu.VMEM_SHARED((ACC_ROWS, feat), jnp.float32),
            pltpu.SemaphoreType.DMA,
        ],
        compiler_params=pltpu.CompilerParams(use_tc_tiling_on_sc=False),
    )
    def edge_agg(z_hbm, src_hbm, dst_hbm, zero_hbm, out_hbm,
                 src_v, dst_v, rows_v, acc_sh, sem):
        cid = lax.axis_index("c")
        sid = lax.axis_index("s")
        wid = cid * NS + sid

        r0 = sid * ROWS_PER_TILE
        pltpu.sync_copy(zero_hbm.at[pl.ds(r0, ROWS_PER_TILE)],
                        acc_sh.at[pl.ds(r0, ROWS_PER_TILE)])
        b0 = wid * blks
        pltpu.sync_copy(src_hbm.at[pl.ds(b0, blks)], src_v)
        pltpu.sync_copy(dst_hbm.at[pl.ds(b0, blks)], dst_v)
        plsc.subcore_barrier()

        pltpu.async_copy(z_hbm.at[src_v.at[0]], rows_v.at[0], sem)

        def body(j, carry):
            b = lax.rem(j, 2)
            pltpu.make_async_copy(z_hbm.at[src_v.at[j]], rows_v.at[b],
                                  sem).wait()

            @pl.when(j + 1 < blks)
            def _():
                pltpu.async_copy(z_hbm.at[src_v.at[j + 1]], rows_v.at[1 - b],
                                 sem)

            pltpu.sync_copy(rows_v.at[b], acc_sh.at[dst_v.at[j]], add=True)
            return carry

        lax.fori_loop(0, blks, body, 0)
        plsc.subcore_barrier()
        pltpu.sync_copy(acc_sh.at[pl.ds(r0, ROWS_PER_TILE)],
                        out_hbm.at[cid, pl.ds(r0, ROWS_PER_TILE)])

    return edge_agg


ROW_BLK = 1000
GRID = N_NODES // ROW_BLK


def _layer1_body(p0_ref, p1_ref, x_ref, wl_ref, wr_ref, b_ref, h_ref):
    agg = p0_ref[0] + p1_ref[0]
    pre = (jnp.dot(agg, wl_ref[...], preferred_element_type=jnp.float32)
           + b_ref[...]
           + jnp.dot(x_ref[...], wr_ref[...],
                     preferred_element_type=jnp.float32))
    h_ref[...] = jnp.maximum(pre, 0.0)


def _layer2_body(q0_ref, q1_ref, h_ref, wl_ref, wr_ref, b_ref, o_ref):
    agg = q0_ref[0] + q1_ref[0]
    v = (jnp.dot(agg, wl_ref[...], preferred_element_type=jnp.float32)
         + b_ref[...]
         + jnp.dot(h_ref[...], wr_ref[...],
                   preferred_element_type=jnp.float32))
    m = jnp.max(v, axis=1, keepdims=True)
    e = jnp.exp(v - m)
    o_ref[...] = e / jnp.sum(e, axis=1, keepdims=True)


def _layer1(p, x, wlT, wrT, b):
    return pl.pallas_call(
        _layer1_body,
        grid=(GRID,),
        in_specs=[
            pl.BlockSpec((1, ROW_BLK, IN_CH), lambda i: (0, i, 0)),
            pl.BlockSpec((1, ROW_BLK, IN_CH), lambda i: (1, i, 0)),
            pl.BlockSpec((ROW_BLK, IN_CH), lambda i: (i, 0)),
            pl.BlockSpec((IN_CH, HIDDEN), lambda i: (0, 0)),
            pl.BlockSpec((IN_CH, HIDDEN), lambda i: (0, 0)),
            pl.BlockSpec((1, HIDDEN), lambda i: (0, 0)),
        ],
        out_specs=pl.BlockSpec((ROW_BLK, HIDDEN), lambda i: (i, 0)),
        out_shape=jax.ShapeDtypeStruct((N_NODES, HIDDEN), jnp.float32),
    )(p, p, x, wlT, wrT, b)


def _layer2(q, h, wlT, wrT, b):
    return pl.pallas_call(
        _layer2_body,
        grid=(GRID,),
        in_specs=[
            pl.BlockSpec((1, ROW_BLK, HIDDEN), lambda i: (0, i, 0)),
            pl.BlockSpec((1, ROW_BLK, HIDDEN), lambda i: (1, i, 0)),
            pl.BlockSpec((ROW_BLK, HIDDEN), lambda i: (i, 0)),
            pl.BlockSpec((HIDDEN, OUT_CH), lambda i: (0, 0)),
            pl.BlockSpec((HIDDEN, OUT_CH), lambda i: (0, 0)),
            pl.BlockSpec((1, OUT_CH), lambda i: (0, 0)),
        ],
        out_specs=pl.BlockSpec((ROW_BLK, OUT_CH), lambda i: (i, 0)),
        out_shape=jax.ShapeDtypeStruct((N_NODES, OUT_CH), jnp.float32),
    )(q, q, h, wlT, wrT, b)


def kernel(x, edge_index, W1_l, b1_l, W1_r, W2_l, b2_l, W2_r):
    src = edge_index[0].astype(jnp.int32)
    dst = edge_index[1].astype(jnp.int32)
    pad = E_PAD - N_EDGES
    src_p = jnp.concatenate([src, jnp.zeros((pad,), jnp.int32)])
    dst_p = jnp.concatenate([dst, jnp.full((pad,), N_NODES, jnp.int32)])

    zero128 = jnp.zeros((ACC_ROWS, IN_CH), jnp.float32)
    p = _make_edge_agg(IN_CH, 64)(x, src_p.reshape(-1, 64),
                                  dst_p.reshape(-1, 64), zero128)
    h = _layer1(p, x, W1_l.T, W1_r.T, b1_l.reshape(1, HIDDEN))
    q = _make_edge_agg(HIDDEN, 64)(h, src_p.reshape(-1, 64),
                                   dst_p.reshape(-1, 64), zero128)
    return _layer2(q, h, W2_l.T, W2_r.T, b2_l.reshape(1, OUT_CH))


# speedup vs baseline: 8.5084x; 2.9576x over previous
import functools

import jax
import jax.numpy as jnp
from jax import lax
from jax.experimental import pallas as pl
from jax.experimental.pallas import tpu as pltpu
from jax.experimental.pallas import tpu_sc as plsc

N_NODES = 10000
N_EDGES = 320000
IN_CH = 128
HIDDEN = 128
OUT_CH = 64
HALF = 64

NC = 2
NS = 16
CHUNK = 64
EPT = 20480
E_PAD = NS * EPT
BLKS = EPT // CHUNK
ACC_ROWS = 10240
ROWS_PER_TILE = ACC_ROWS // NS
ZROWS_PER_TILE = N_NODES // NS


def _make_edge_agg():
    mesh = plsc.VectorSubcoreMesh(core_axis_name="c", subcore_axis_name="s",
                                  num_cores=NC, num_subcores=NS)

    @functools.partial(
        pl.kernel,
        out_type=jax.ShapeDtypeStruct((NC, ACC_ROWS, HALF), jnp.float32),
        mesh=mesh,
        scratch_types=[
            pltpu.VMEM((BLKS, CHUNK), jnp.int32),
            pltpu.VMEM((BLKS, CHUNK), jnp.int32),
            pltpu.VMEM((2, CHUNK, HALF), jnp.float32),
            pltpu.VMEM_SHARED((N_NODES, HALF), jnp.float32),
            pltpu.VMEM_SHARED((ACC_ROWS, HALF), jnp.float32),
            pltpu.SemaphoreType.DMA,
        ],
        compiler_params=pltpu.CompilerParams(use_tc_tiling_on_sc=False),
    )
    def edge_agg(z_hbm, src_hbm, dst_hbm, zero_hbm, out_hbm,
                 src_v, dst_v, rows_v, z_sh, acc_sh, sem):
        cid = lax.axis_index("c")
        sid = lax.axis_index("s")

        zr0 = sid * ZROWS_PER_TILE
        pltpu.sync_copy(z_hbm.at[cid, pl.ds(zr0, ZROWS_PER_TILE)],
                        z_sh.at[pl.ds(zr0, ZROWS_PER_TILE)])
        r0 = sid * ROWS_PER_TILE
        pltpu.sync_copy(zero_hbm.at[pl.ds(r0, ROWS_PER_TILE)],
                        acc_sh.at[pl.ds(r0, ROWS_PER_TILE)])
        b0 = sid * BLKS
        pltpu.sync_copy(src_hbm.at[pl.ds(b0, BLKS)], src_v)
        pltpu.sync_copy(dst_hbm.at[pl.ds(b0, BLKS)], dst_v)
        plsc.subcore_barrier()

        pltpu.async_copy(z_sh.at[src_v.at[0]], rows_v.at[0], sem)

        def body(j, carry):
            b = lax.rem(j, 2)
            pltpu.make_async_copy(z_sh.at[src_v.at[j]], rows_v.at[b],
                                  sem).wait()

            @pl.when(j + 1 < BLKS)
            def _():
                pltpu.async_copy(z_sh.at[src_v.at[j + 1]], rows_v.at[1 - b],
                                 sem)

            pltpu.sync_copy(rows_v.at[b], acc_sh.at[dst_v.at[j]], add=True)
            return carry

        lax.fori_loop(0, BLKS, body, 0)
        plsc.subcore_barrier()
        pltpu.sync_copy(acc_sh.at[pl.ds(r0, ROWS_PER_TILE)],
                        out_hbm.at[cid, pl.ds(r0, ROWS_PER_TILE)])

    return edge_agg


_EDGE_AGG = [None]


def _edge_agg():
    if _EDGE_AGG[0] is None:
        _EDGE_AGG[0] = _make_edge_agg()
    return _EDGE_AGG[0]


ROW_BLK = 1000
GRID = N_NODES // ROW_BLK


def _layer1_body(p0_ref, p1_ref, x_ref, wl_ref, wr_ref, b_ref, h_ref):
    agg = jnp.concatenate([p0_ref[0], p1_ref[0]], axis=1)
    pre = (jnp.dot(agg, wl_ref[...], preferred_element_type=jnp.float32)
           + b_ref[...]
           + jnp.dot(x_ref[...], wr_ref[...],
                     preferred_element_type=jnp.float32))
    h = jnp.maximum(pre, 0.0)
    h_ref[...] = jnp.stack([h[:, :HALF], h[:, HALF:]])


def _layer2_body(q0_ref, q1_ref, h0_ref, h1_ref, wl_ref, wr_ref, b_ref,
                 o_ref):
    agg = jnp.concatenate([q0_ref[0], q1_ref[0]], axis=1)
    hv = jnp.concatenate([h0_ref[0], h1_ref[0]], axis=1)
    v = (jnp.dot(agg, wl_ref[...], preferred_element_type=jnp.float32)
         + b_ref[...]
         + jnp.dot(hv, wr_ref[...], preferred_element_type=jnp.float32))
    m = jnp.max(v, axis=1, keepdims=True)
    e = jnp.exp(v - m)
    o_ref[...] = e / jnp.sum(e, axis=1, keepdims=True)


def _layer1(p, x, wlT, wrT, b):
    return pl.pallas_call(
        _layer1_body,
        grid=(GRID,),
        in_specs=[
            pl.BlockSpec((1, ROW_BLK, HALF), lambda i: (0, i, 0)),
            pl.BlockSpec((1, ROW_BLK, HALF), lambda i: (1, i, 0)),
            pl.BlockSpec((ROW_BLK, IN_CH), lambda i: (i, 0)),
            pl.BlockSpec((IN_CH, HIDDEN), lambda i: (0, 0)),
            pl.BlockSpec((IN_CH, HIDDEN), lambda i: (0, 0)),
            pl.BlockSpec((1, HIDDEN), lambda i: (0, 0)),
        ],
        out_specs=pl.BlockSpec((2, ROW_BLK, HALF), lambda i: (0, i, 0)),
        out_shape=jax.ShapeDtypeStruct((2, N_NODES, HALF), jnp.float32),
    )(p, p, x, wlT, wrT, b)


def _layer2(q, h, wlT, wrT, b):
    return pl.pallas_call(
        _layer2_body,
        grid=(GRID,),
        in_specs=[
            pl.BlockSpec((1, ROW_BLK, HALF), lambda i: (0, i, 0)),
            pl.BlockSpec((1, ROW_BLK, HALF), lambda i: (1, i, 0)),
            pl.BlockSpec((1, ROW_BLK, HALF), lambda i: (0, i, 0)),
            pl.BlockSpec((1, ROW_BLK, HALF), lambda i: (1, i, 0)),
            pl.BlockSpec((HIDDEN, OUT_CH), lambda i: (0, 0)),
            pl.BlockSpec((HIDDEN, OUT_CH), lambda i: (0, 0)),
            pl.BlockSpec((1, OUT_CH), lambda i: (0, 0)),
        ],
        out_specs=pl.BlockSpec((ROW_BLK, OUT_CH), lambda i: (i, 0)),
        out_shape=jax.ShapeDtypeStruct((N_NODES, OUT_CH), jnp.float32),
    )(q, q, h, h, wlT, wrT, b)


def kernel(x, edge_index, W1_l, b1_l, W1_r, W2_l, b2_l, W2_r):
    src = edge_index[0].astype(jnp.int32)
    dst = edge_index[1].astype(jnp.int32)
    pad = E_PAD - N_EDGES
    src_p = jnp.concatenate([src, jnp.zeros((pad,), jnp.int32)])
    dst_p = jnp.concatenate([dst, jnp.full((pad,), N_NODES, jnp.int32)])
    src_p = src_p.reshape(-1, CHUNK)
    dst_p = dst_p.reshape(-1, CHUNK)

    xs = jnp.stack([x[:, :HALF], x[:, HALF:]])
    zero = jnp.zeros((ACC_ROWS, HALF), jnp.float32)
    agg = _edge_agg()
    p = agg(xs, src_p, dst_p, zero)
    h = _layer1(p, x, W1_l.T, W1_r.T, b1_l.reshape(1, HIDDEN))
    q = agg(h, src_p, dst_p, zero)
    return _layer2(q, h, W2_l.T, W2_r.T, b2_l.reshape(1, OUT_CH))


# speedup vs baseline: 9.5099x; 1.1177x over previous
import functools

import jax
import jax.numpy as jnp
from jax import lax
from jax.experimental import pallas as pl
from jax.experimental.pallas import tpu as pltpu
from jax.experimental.pallas import tpu_sc as plsc

N_NODES = 10000
N_EDGES = 320000
IN_CH = 128
HIDDEN = 128
OUT_CH = 64
HALF = 64

NC = 2
NS = 16
CHUNK = 64
EPT = 20032
E_PAD = NS * EPT
BLKS = EPT // CHUNK
ACC_ROWS = 10240
ROWS_PER_TILE = ACC_ROWS // NS
ZROWS_PER_TILE = N_NODES // NS


def _make_edge_agg():
    mesh = plsc.VectorSubcoreMesh(core_axis_name="c", subcore_axis_name="s",
                                  num_cores=NC, num_subcores=NS)

    @functools.partial(
        pl.kernel,
        out_type=jax.ShapeDtypeStruct((NC, ACC_ROWS, HALF), jnp.float32),
        mesh=mesh,
        scratch_types=[
            pltpu.VMEM((BLKS, CHUNK), jnp.int32),
            pltpu.VMEM((BLKS, CHUNK), jnp.int32),
            pltpu.VMEM((2, CHUNK, HALF), jnp.float32),
            pltpu.VMEM_SHARED((N_NODES, HALF), jnp.float32),
            pltpu.VMEM_SHARED((ACC_ROWS, HALF), jnp.float32),
            pltpu.SemaphoreType.DMA,
        ],
        compiler_params=pltpu.CompilerParams(use_tc_tiling_on_sc=False),
    )
    def edge_agg(z_hbm, src_hbm, dst_hbm, zero_hbm, out_hbm,
                 src_v, dst_v, rows_v, z_sh, acc_sh, sem):
        cid = lax.axis_index("c")
        sid = lax.axis_index("s")

        zr0 = sid * ZROWS_PER_TILE
        pltpu.sync_copy(
            z_hbm.at[pl.ds(zr0, ZROWS_PER_TILE), pl.ds(cid * HALF, HALF)],
            z_sh.at[pl.ds(zr0, ZROWS_PER_TILE)])
        r0 = sid * ROWS_PER_TILE
        pltpu.sync_copy(zero_hbm.at[pl.ds(r0, ROWS_PER_TILE)],
                        acc_sh.at[pl.ds(r0, ROWS_PER_TILE)])
        b0 = sid * BLKS
        pltpu.sync_copy(src_hbm.at[pl.ds(b0, BLKS)], src_v)
        pltpu.sync_copy(dst_hbm.at[pl.ds(b0, BLKS)], dst_v)
        plsc.subcore_barrier()

        pltpu.async_copy(z_sh.at[src_v.at[0]], rows_v.at[0], sem)

        def body(j, carry):
            b = lax.rem(j, 2)
            pltpu.make_async_copy(z_sh.at[src_v.at[j]], rows_v.at[b],
                                  sem).wait()

            @pl.when(j + 1 < BLKS)
            def _():
                pltpu.async_copy(z_sh.at[src_v.at[j + 1]], rows_v.at[1 - b],
                                 sem)

            pltpu.sync_copy(rows_v.at[b], acc_sh.at[dst_v.at[j]], add=True)
            return carry

        lax.fori_loop(0, BLKS, body, 0)
        plsc.subcore_barrier()
        pltpu.sync_copy(acc_sh.at[pl.ds(r0, ROWS_PER_TILE)],
                        out_hbm.at[cid, pl.ds(r0, ROWS_PER_TILE)])

    return edge_agg


_EDGE_AGG = [None]


def _edge_agg():
    if _EDGE_AGG[0] is None:
        _EDGE_AGG[0] = _make_edge_agg()
    return _EDGE_AGG[0]


ROW_BLK = 1000
GRID = N_NODES // ROW_BLK


def _layer1_body(p0_ref, p1_ref, x_ref, wl_ref, wr_ref, b_ref, h_ref):
    agg = jnp.concatenate([p0_ref[0], p1_ref[0]], axis=1)
    pre = (jnp.dot(agg, wl_ref[...], preferred_element_type=jnp.float32)
           + b_ref[...]
           + jnp.dot(x_ref[...], wr_ref[...],
                     preferred_element_type=jnp.float32))
    h_ref[...] = jnp.maximum(pre, 0.0)


def _layer2_body(q0_ref, q1_ref, h_ref, wl_ref, wr_ref, b_ref, o_ref):
    agg = jnp.concatenate([q0_ref[0], q1_ref[0]], axis=1)
    v = (jnp.dot(agg, wl_ref[...], preferred_element_type=jnp.float32)
         + b_ref[...]
         + jnp.dot(h_ref[...], wr_ref[...],
                   preferred_element_type=jnp.float32))
    m = jnp.max(v, axis=1, keepdims=True)
    e = jnp.exp(v - m)
    o_ref[...] = e / jnp.sum(e, axis=1, keepdims=True)


def _layer1(p, x, wlT, wrT, b):
    return pl.pallas_call(
        _layer1_body,
        grid=(GRID,),
        in_specs=[
            pl.BlockSpec((1, ROW_BLK, HALF), lambda i: (0, i, 0)),
            pl.BlockSpec((1, ROW_BLK, HALF), lambda i: (1, i, 0)),
            pl.BlockSpec((ROW_BLK, IN_CH), lambda i: (i, 0)),
            pl.BlockSpec((IN_CH, HIDDEN), lambda i: (0, 0)),
            pl.BlockSpec((IN_CH, HIDDEN), lambda i: (0, 0)),
            pl.BlockSpec((1, HIDDEN), lambda i: (0, 0)),
        ],
        out_specs=pl.BlockSpec((ROW_BLK, HIDDEN), lambda i: (i, 0)),
        out_shape=jax.ShapeDtypeStruct((N_NODES, HIDDEN), jnp.float32),
    )(p, p, x, wlT, wrT, b)


def _layer2(q, h, wlT, wrT, b):
    return pl.pallas_call(
        _layer2_body,
        grid=(GRID,),
        in_specs=[
            pl.BlockSpec((1, ROW_BLK, HALF), lambda i: (0, i, 0)),
            pl.BlockSpec((1, ROW_BLK, HALF), lambda i: (1, i, 0)),
            pl.BlockSpec((ROW_BLK, HIDDEN), lambda i: (i, 0)),
            pl.BlockSpec((HIDDEN, OUT_CH), lambda i: (0, 0)),
            pl.BlockSpec((HIDDEN, OUT_CH), lambda i: (0, 0)),
            pl.BlockSpec((1, OUT_CH), lambda i: (0, 0)),
        ],
        out_specs=pl.BlockSpec((ROW_BLK, OUT_CH), lambda i: (i, 0)),
        out_shape=jax.ShapeDtypeStruct((N_NODES, OUT_CH), jnp.float32),
    )(q, q, h, wlT, wrT, b)


def kernel(x, edge_index, W1_l, b1_l, W1_r, W2_l, b2_l, W2_r):
    src = edge_index[0].astype(jnp.int32)
    dst = edge_index[1].astype(jnp.int32)
    pad = E_PAD - N_EDGES
    src_p = jnp.concatenate([src, jnp.zeros((pad,), jnp.int32)])
    dst_p = jnp.concatenate([dst, jnp.full((pad,), N_NODES, jnp.int32)])
    src_p = src_p.reshape(-1, CHUNK)
    dst_p = dst_p.reshape(-1, CHUNK)

    zero = jnp.zeros((ACC_ROWS, HALF), jnp.float32)
    agg = _edge_agg()
    p = agg(x, src_p, dst_p, zero)
    h = _layer1(p, x, W1_l.T, W1_r.T, b1_l.reshape(1, HIDDEN))
    q = agg(h, src_p, dst_p, zero)
    return _layer2(q, h, W2_l.T, W2_r.T, b2_l.reshape(1, OUT_CH))
